# Initial kernel scaffold; baseline (speedup 1.0000x reference)
import jax
import jax.numpy as jnp
from jax.experimental import pallas as pl


def kernel(feats, points, params):
    raise NotImplementedError("write your pallas kernel here")


# speedup vs baseline: 13.3892x; 13.3892x over previous
import functools

import jax
import jax.numpy as jnp
import numpy as np
from jax import lax
from jax.experimental import pallas as pl
from jax.experimental.pallas import tpu as pltpu
from jax.experimental.pallas import tpu_sc as plsc

_K = 16
_EPS = 1e-5

_SQRT_HALF = np.float32(1.0 / np.sqrt(2.0))


def _gelu(x):
    return 0.5 * x * (1.0 + lax.erf(x * _SQRT_HALF))


def _fps_body(xyz_ref, idx_ref, pts_ref):
    X = xyz_ref[:, 0]
    Y = xyz_ref[:, 1]
    Z = xyz_ref[:, 2]
    B, S, L = X.shape
    n2 = idx_ref.shape[0]
    flat = (lax.broadcasted_iota(jnp.int32, (B, S, L), 1) * L
            + lax.broadcasted_iota(jnp.int32, (B, S, L), 2))
    BIG = jnp.int32(1 << 30)

    def step(t, carry):
        D, cur = carry
        sel = flat == cur
        px = jnp.max(jnp.where(sel, X, -jnp.inf), axis=2, keepdims=True).max(
            axis=1, keepdims=True)
        py = jnp.max(jnp.where(sel, Y, -jnp.inf), axis=2, keepdims=True).max(
            axis=1, keepdims=True)
        pz = jnp.max(jnp.where(sel, Z, -jnp.inf), axis=2, keepdims=True).max(
            axis=1, keepdims=True)
        dx = X - px
        dy = Y - py
        dz = Z - pz
        dist = dx * dx + dy * dy + dz * dz
        D = jnp.minimum(D, dist)
        m = jnp.max(D, axis=2, keepdims=True).max(axis=1, keepdims=True)
        nxt = jnp.min(jnp.where(D == m, flat, BIG), axis=2, keepdims=True).min(
            axis=1, keepdims=True)
        idx_ref[pl.ds(t, 1), :] = cur.reshape(1, B)
        pts_ref[pl.ds(t, 1), :] = jnp.concatenate([px, py, pz], axis=1).reshape(
            1, 3 * B)
        return D, nxt

    D0 = jnp.full((B, S, L), 1e10, jnp.float32)
    cur0 = jnp.zeros((B, 1, 1), jnp.int32)
    lax.fori_loop(0, n2, step, (D0, cur0))


def _fps(points, n2):
    b, n, _ = points.shape
    S, L = n // 512, 512
    xyz = points.transpose(0, 2, 1).reshape(b, 3, S, L)
    idx2d, pts2d = pl.pallas_call(
        _fps_body,
        out_shape=[
            jax.ShapeDtypeStruct((n2, b), jnp.int32),
            jax.ShapeDtypeStruct((n2, 3 * b), jnp.float32),
        ],
    )(xyz)
    p2_idx = idx2d.T
    p2_points = pts2d.reshape(n2, b, 3).transpose(1, 0, 2)
    return p2_idx, p2_points


def _knn_body(q_ref, rt_ref, idx_ref):
    q = q_ref[0]
    rt = rt_ref[0]
    qn = jnp.sum(q * q, axis=1, keepdims=True)
    rn = jnp.sum(rt * rt, axis=0, keepdims=True)
    qr = jnp.dot(q.astype(jnp.bfloat16), rt.astype(jnp.bfloat16),
                 preferred_element_type=jnp.float32)
    D = (qn + rn) - 2.0 * qr
    lane = lax.broadcasted_iota(jnp.int32, D.shape, 1)
    BIGI = jnp.int32(1 << 30)
    BIGF = jnp.float32(3e38)
    cols = []
    for _ in range(_K):
        m = jnp.min(D, axis=1, keepdims=True)
        j = jnp.min(jnp.where(D == m, lane, BIGI), axis=1, keepdims=True)
        cols.append(j)
        D = jnp.where(lane == j, BIGF, D)
    idx_ref[0] = jnp.concatenate(cols, axis=1)


def _knn(query, ref):
    b, nq, _ = query.shape
    nr = ref.shape[1]
    QT = 256
    qpad = jnp.pad(query, ((0, 0), (0, 0), (0, 5)))
    rt = jnp.pad(ref.transpose(0, 2, 1), ((0, 0), (0, 5), (0, 0)))
    grid = (b, nq // QT)
    return pl.pallas_call(
        _knn_body,
        grid=grid,
        in_specs=[
            pl.BlockSpec((1, QT, 8), lambda i, j: (i, j, 0)),
            pl.BlockSpec((1, 8, nr), lambda i, j: (i, 0, 0)),
        ],
        out_specs=pl.BlockSpec((1, QT, _K), lambda i, j: (i, j, 0)),
        out_shape=jax.ShapeDtypeStruct((b, nq, _K), jnp.int32),
    )(qpad, rt)


@functools.partial(jax.jit, static_argnums=(2, 3, 4))
def _gather_rows(table, idx, R, D, NF):
    info = plsc.get_sparse_core_info()
    NW = info.num_cores * info.num_subcores
    b_per_w = NF // NW
    CH = 8
    while CH * 2 <= b_per_w and CH * 2 * D <= 65536:
        CH *= 2
    n_ch = b_per_w // CH
    mesh = plsc.VectorSubcoreMesh(core_axis_name="c", subcore_axis_name="s")

    @functools.partial(
        pl.kernel,
        mesh=mesh,
        out_type=jax.ShapeDtypeStruct((NF, D), jnp.float32),
        scratch_types=[
            pltpu.VMEM((CH,), jnp.int32),
            pltpu.VMEM((CH, D), jnp.float32),
            pltpu.SemaphoreType.DMA,
        ],
    )
    def gk(table_hbm, idx_hbm, out_hbm, idx_v, rows_v, sem):
        wid = lax.axis_index("s") * info.num_cores + lax.axis_index("c")
        base = wid * b_per_w

        def chunk(i, c):
            off = base + i * CH
            pltpu.sync_copy(idx_hbm.at[pl.ds(off, CH)], idx_v)
            pltpu.async_copy(table_hbm.at[idx_v], rows_v, sem).wait()
            pltpu.sync_copy(rows_v, out_hbm.at[pl.ds(off, CH)])
            return c

        lax.fori_loop(0, n_ch, chunk, 0)

    return gk(table, idx)


def _t1_body(p_ref, f_ref, wp_ref, wf_ref, b_ref, o_ref):
    acc = jnp.dot(p_ref[0], wp_ref[...], preferred_element_type=jnp.float32)
    acc = acc + jnp.dot(f_ref[0], wf_ref[...], preferred_element_type=jnp.float32)
    o_ref[0] = acc + b_ref[...]


def _stats_body(x_ref, o_ref, acc_ref):
    i = pl.program_id(0)
    x = x_ref[...]
    s = jnp.sum(x, axis=0, keepdims=True)
    ss = jnp.sum(x * x, axis=0, keepdims=True)

    @pl.when(i == 0)
    def _():
        acc_ref[...] = jnp.zeros_like(acc_ref)

    acc_ref[0:1, :] += s
    acc_ref[1:2, :] += ss

    @pl.when(i == pl.num_programs(0) - 1)
    def _():
        o_ref[...] = acc_ref[...]


def _stats(x_flat, tile):
    n, d = x_flat.shape
    return pl.pallas_call(
        _stats_body,
        grid=(n // tile,),
        in_specs=[pl.BlockSpec((tile, d), lambda i: (i, 0))],
        out_specs=pl.BlockSpec((8, d), lambda i: (0, 0)),
        out_shape=jax.ShapeDtypeStruct((8, d), jnp.float32),
        scratch_shapes=[pltpu.VMEM((8, d), jnp.float32)],
    )(x_flat)


def _bn_from_stats(x, st, g, b, count):
    m = st[0:1, :] / count
    v = st[1:2, :] / count - m * m
    return g * (x - m) * lax.rsqrt(v + _EPS) + b


def _td_apply_body(x_ref, st_ref, g_ref, b_ref, o_ref, *, count):
    y = _gelu(_bn_from_stats(x_ref[...], st_ref[...], g_ref[...], b_ref[...],
                             count))
    nq = o_ref.shape[0]
    o_ref[...] = jnp.max(y.reshape(nq, _K, y.shape[-1]), axis=1)


def _fc1qkv_body(f_ref, w1_ref, b1_ref, g1_ref, gb1_ref, qw_ref, qb_ref,
                 kw_ref, kb_ref, vw_ref, vb_ref, q_ref, kv_ref):
    a = jnp.dot(f_ref[...], w1_ref[...],
                preferred_element_type=jnp.float32) + b1_ref[...]
    n = a.shape[0]
    m = jnp.sum(a, axis=0, keepdims=True) / n
    d = a - m
    v = jnp.sum(d * d, axis=0, keepdims=True) / n
    f1 = _gelu(g1_ref[...] * d * lax.rsqrt(v + _EPS) + gb1_ref[...])
    q_ref[...] = jnp.dot(f1, qw_ref[...],
                         preferred_element_type=jnp.float32) + qb_ref[...]
    kv_ref[:, 0:128] = jnp.dot(f1, kw_ref[...],
                               preferred_element_type=jnp.float32) + kb_ref[...]
    kv_ref[:, 128:256] = jnp.dot(f1, vw_ref[...],
                                 preferred_element_type=jnp.float32) + vb_ref[...]


def _p1_body(qp_ref, kp_ref, w_ref, b_ref, pre_ref, st_ref, acc_ref):
    i = pl.program_id(0)
    nq = qp_ref.shape[0]
    qpb = jnp.broadcast_to(qp_ref[...][:, None, :],
                           (nq, _K, 16)).reshape(nq * _K, 16)
    dp = qpb - kp_ref[:, 256:272]
    pre = jnp.dot(dp, w_ref[...],
                  preferred_element_type=jnp.float32) + b_ref[...]
    pre_ref[...] = pre

    @pl.when(i == 0)
    def _():
        acc_ref[...] = jnp.zeros_like(acc_ref)

    acc_ref[0:1, :] += jnp.sum(pre, axis=0, keepdims=True)
    acc_ref[1:2, :] += jnp.sum(pre * pre, axis=0, keepdims=True)

    @pl.when(i == pl.num_programs(0) - 1)
    def _():
        st_ref[...] = acc_ref[...]


def _p2_body(pre_ref, st_ref, bg_ref, bb_ref, w2_ref, b2_ref, q_ref, kk_ref,
             pos_ref, ga_ref, st2_ref, acc_ref, *, count):
    i = pl.program_id(0)
    pos3 = _gelu(_bn_from_stats(pre_ref[...], st_ref[...], bg_ref[...],
                                bb_ref[...], count))
    pos = jnp.dot(pos3, w2_ref[...],
                  preferred_element_type=jnp.float32) + b2_ref[...]
    nq = q_ref.shape[0]
    qb = jnp.broadcast_to(q_ref[...][:, None, :],
                          (nq, _K, 128)).reshape(nq * _K, 128)
    ga = (qb - kk_ref[:, 0:128]) + pos
    pos_ref[...] = pos
    ga_ref[...] = ga

    @pl.when(i == 0)
    def _():
        acc_ref[...] = jnp.zeros_like(acc_ref)

    acc_ref[0:1, :] += jnp.sum(ga, axis=0, keepdims=True)
    acc_ref[1:2, :] += jnp.sum(ga * ga, axis=0, keepdims=True)

    @pl.when(i == pl.num_programs(0) - 1)
    def _():
        st2_ref[...] = acc_ref[...]


def _p3_body(ga_ref, st_ref, bg_ref, bb_ref, w_ref, b_ref, gb_ref, st2_ref,
             acc_ref, *, count):
    i = pl.program_id(0)
    h = _gelu(_bn_from_stats(ga_ref[...], st_ref[...], bg_ref[...],
                             bb_ref[...], count))
    gb = jnp.dot(h, w_ref[...], preferred_element_type=jnp.float32) + b_ref[...]
    gb_ref[...] = gb

    @pl.when(i == 0)
    def _():
        acc_ref[...] = jnp.zeros_like(acc_ref)

    acc_ref[0:1, :] += jnp.sum(gb, axis=0, keepdims=True)
    acc_ref[1:2, :] += jnp.sum(gb * gb, axis=0, keepdims=True)

    @pl.when(i == pl.num_programs(0) - 1)
    def _():
        st2_ref[...] = acc_ref[...]


def _p4_body(gb_ref, st_ref, bg_ref, bb_ref, w_ref, b_ref, kv_ref, pos_ref,
             o_ref, *, count):
    h = _gelu(_bn_from_stats(gb_ref[...], st_ref[...], bg_ref[...],
                             bb_ref[...], count))
    gam = jnp.dot(h, w_ref[...],
                  preferred_element_type=jnp.float32) + b_ref[...]
    nq = o_ref.shape[0]
    G = gam.reshape(nq, _K, 128)
    mx = jnp.max(G, axis=1, keepdims=True)
    e = jnp.exp(G - mx)
    s = jnp.sum(e, axis=1, keepdims=True)
    rho = e / s
    val = (kv_ref[:, 128:256] + pos_ref[...]).reshape(nq, _K, 128)
    o_ref[...] = jnp.sum(rho * val, axis=1)


def _out_body(x_ref, ftd_ref, g2_ref, b2_ref, w_ref, bw_ref, g3_ref, b3_ref,
              o_ref):
    x = x_ref[...]
    n = x.shape[0]
    m = jnp.sum(x, axis=0, keepdims=True) / n
    d = x - m
    v = jnp.sum(d * d, axis=0, keepdims=True) / n
    a = _gelu(g2_ref[...] * d * lax.rsqrt(v + _EPS) + b2_ref[...])
    f2 = jnp.dot(a, w_ref[...], preferred_element_type=jnp.float32) + bw_ref[...]
    m2 = jnp.sum(f2, axis=0, keepdims=True) / n
    d2 = f2 - m2
    v2 = jnp.sum(d2 * d2, axis=0, keepdims=True) / n
    a2 = _gelu(g3_ref[...] * d2 * lax.rsqrt(v2 + _EPS) + b3_ref[...])
    o_ref[...] = ftd_ref[...] + a2


def kernel(feats, points, params):
    b, n, df = feats.shape
    n2 = n // 4
    nf = b * n2 * _K
    row1 = lambda a: a.reshape(1, -1)

    p2_idx, p2_points = _fps(points, n2)

    knn1 = _knn(p2_points, points)
    idx1 = (knn1 + (jnp.arange(b, dtype=jnp.int32) * n)[:, None, None]
            ).reshape(-1)

    pts_pad = jnp.pad(points, ((0, 0), (0, 0), (0, 5)))
    wp = jnp.pad(params['td_fc_W'][:3], ((0, 5), (0, 0)))
    wf = params['td_fc_W'][3:]
    t1 = pl.pallas_call(
        _t1_body,
        grid=(b,),
        in_specs=[
            pl.BlockSpec((1, n, 8), lambda i: (i, 0, 0)),
            pl.BlockSpec((1, n, df), lambda i: (i, 0, 0)),
            pl.BlockSpec((8, 128), lambda i: (0, 0)),
            pl.BlockSpec((df, 128), lambda i: (0, 0)),
            pl.BlockSpec((1, 128), lambda i: (0, 0)),
        ],
        out_specs=pl.BlockSpec((1, n, 128), lambda i: (i, 0, 0)),
        out_shape=jax.ShapeDtypeStruct((b, n, 128), jnp.float32),
    )(pts_pad, feats, wp, wf, row1(params['td_fc_b']))
    t1f = t1.reshape(b * n, 128)

    x_rows = _gather_rows(t1f, idx1, b * n, 128, nf)

    st_td = _stats(x_rows, 2048)
    f_td = pl.pallas_call(
        functools.partial(_td_apply_body, count=float(nf)),
        grid=(nf // 2048,),
        in_specs=[
            pl.BlockSpec((2048, 128), lambda i: (i, 0)),
            pl.BlockSpec((8, 128), lambda i: (0, 0)),
            pl.BlockSpec((1, 128), lambda i: (0, 0)),
            pl.BlockSpec((1, 128), lambda i: (0, 0)),
        ],
        out_specs=pl.BlockSpec((128, 128), lambda i: (i, 0)),
        out_shape=jax.ShapeDtypeStruct((b * n2, 128), jnp.float32),
    )(x_rows, st_td, row1(params['td_bn_g']), row1(params['td_bn_b']))

    q_tab, kv_tab = pl.pallas_call(
        _fc1qkv_body,
        out_shape=[
            jax.ShapeDtypeStruct((b * n2, 128), jnp.float32),
            jax.ShapeDtypeStruct((b * n2, 256), jnp.float32),
        ],
    )(f_td, params['fc1_W'], row1(params['fc1_b']), row1(params['bn1_g']),
      row1(params['bn1_b']), params['q_W'], row1(params['q_b']),
      params['k_W'], row1(params['k_b']), params['v_W'], row1(params['v_b']))

    knn2 = _knn(p2_points, p2_points)
    idx2 = (knn2 + (jnp.arange(b, dtype=jnp.int32) * n2)[:, None, None]
            ).reshape(-1)
    p2f = p2_points.reshape(b * n2, 3)
    p2pad = jnp.pad(p2f, ((0, 0), (0, 13)))
    kvp_tab = jnp.concatenate(
        [kv_tab, jnp.pad(p2f, ((0, 0), (0, 125)))], axis=1)
    kv_rows = _gather_rows(kvp_tab, idx2, b * n2, 384, nf)

    TQ = 256
    TR = TQ * _K
    grid = ((b * n2) // TQ,)
    d1w = jnp.pad(params['d1_W'], ((0, 13), (0, 13)))
    d1b = jnp.pad(params['d1_b'], (0, 13)).reshape(1, 16)
    bndg = jnp.pad(params['bnd_g'], (0, 13)).reshape(1, 16)
    bndb = jnp.pad(params['bnd_b'], (0, 13)).reshape(1, 16)
    d2w = jnp.pad(params['d2_W'], ((0, 13), (0, 0)))

    pre, st_d = pl.pallas_call(
        _p1_body,
        grid=grid,
        in_specs=[
            pl.BlockSpec((TQ, 16), lambda i: (i, 0)),
            pl.BlockSpec((TR, 384), lambda i: (i, 0)),
            pl.BlockSpec((16, 16), lambda i: (0, 0)),
            pl.BlockSpec((1, 16), lambda i: (0, 0)),
        ],
        out_specs=[
            pl.BlockSpec((TR, 16), lambda i: (i, 0)),
            pl.BlockSpec((8, 16), lambda i: (0, 0)),
        ],
        out_shape=[
            jax.ShapeDtypeStruct((nf, 16), jnp.float32),
            jax.ShapeDtypeStruct((8, 16), jnp.float32),
        ],
        scratch_shapes=[pltpu.VMEM((8, 16), jnp.float32)],
    )(p2pad, kv_rows, d1w, d1b)

    pos, gam_a, st_g1 = pl.pallas_call(
        functools.partial(_p2_body, count=float(nf)),
        grid=grid,
        in_specs=[
            pl.BlockSpec((TR, 16), lambda i: (i, 0)),
            pl.BlockSpec((8, 16), lambda i: (0, 0)),
            pl.BlockSpec((1, 16), lambda i: (0, 0)),
            pl.BlockSpec((1, 16), lambda i: (0, 0)),
            pl.BlockSpec((16, 128), lambda i: (0, 0)),
            pl.BlockSpec((1, 128), lambda i: (0, 0)),
            pl.BlockSpec((TQ, 128), lambda i: (i, 0)),
            pl.BlockSpec((TR, 384), lambda i: (i, 0)),
        ],
        out_specs=[
            pl.BlockSpec((TR, 128), lambda i: (i, 0)),
            pl.BlockSpec((TR, 128), lambda i: (i, 0)),
            pl.BlockSpec((8, 128), lambda i: (0, 0)),
        ],
        out_shape=[
            jax.ShapeDtypeStruct((nf, 128), jnp.float32),
            jax.ShapeDtypeStruct((nf, 128), jnp.float32),
            jax.ShapeDtypeStruct((8, 128), jnp.float32),
        ],
        scratch_shapes=[pltpu.VMEM((8, 128), jnp.float32)],
    )(pre, st_d, bndg, bndb, d2w, row1(params['d2_b']), q_tab, kv_rows)

    gam_b, st_g2 = pl.pallas_call(
        functools.partial(_p3_body, count=float(nf)),
        grid=grid,
        in_specs=[
            pl.BlockSpec((TR, 128), lambda i: (i, 0)),
            pl.BlockSpec((8, 128), lambda i: (0, 0)),
            pl.BlockSpec((1, 128), lambda i: (0, 0)),
            pl.BlockSpec((1, 128), lambda i: (0, 0)),
            pl.BlockSpec((128, 128), lambda i: (0, 0)),
            pl.BlockSpec((1, 128), lambda i: (0, 0)),
        ],
        out_specs=[
            pl.BlockSpec((TR, 128), lambda i: (i, 0)),
            pl.BlockSpec((8, 128), lambda i: (0, 0)),
        ],
        out_shape=[
            jax.ShapeDtypeStruct((nf, 128), jnp.float32),
            jax.ShapeDtypeStruct((8, 128), jnp.float32),
        ],
        scratch_shapes=[pltpu.VMEM((8, 128), jnp.float32)],
    )(gam_a, st_g1, row1(params['bng1_g']), row1(params['bng1_b']),
      params['g1_W'], row1(params['g1_b']))

    attn = pl.pallas_call(
        functools.partial(_p4_body, count=float(nf)),
        grid=grid,
        in_specs=[
            pl.BlockSpec((TR, 128), lambda i: (i, 0)),
            pl.BlockSpec((8, 128), lambda i: (0, 0)),
            pl.BlockSpec((1, 128), lambda i: (0, 0)),
            pl.BlockSpec((1, 128), lambda i: (0, 0)),
            pl.BlockSpec((128, 128), lambda i: (0, 0)),
            pl.BlockSpec((1, 128), lambda i: (0, 0)),
            pl.BlockSpec((TR, 384), lambda i: (i, 0)),
            pl.BlockSpec((TR, 128), lambda i: (i, 0)),
        ],
        out_specs=pl.BlockSpec((TQ, 128), lambda i: (i, 0)),
        out_shape=jax.ShapeDtypeStruct((b * n2, 128), jnp.float32),
    )(gam_b, st_g2, row1(params['bng2_g']), row1(params['bng2_b']),
      params['g2_W'], row1(params['g2_b']), kv_rows, pos)

    out = pl.pallas_call(
        _out_body,
        out_shape=jax.ShapeDtypeStruct((b * n2, 128), jnp.float32),
    )(attn, f_td, row1(params['bn2_g']), row1(params['bn2_b']),
      params['fc2_W'], row1(params['fc2_b']), row1(params['bn3_g']),
      row1(params['bn3_b']))

    return out.reshape(b, n2, 128), p2_points
